# baseline (speedup 1.0000x reference)
import jax
import jax.numpy as jnp
from jax import lax
from jax.experimental import pallas as pl
from jax.experimental.pallas import tpu as pltpu

_KS0 = 0
_KS1 = 12345
_KS2 = _KS0 ^ _KS1 ^ 0x1BD11BDA
_ROTS = ((13, 15, 26, 6), (17, 29, 16, 24))

_ROWS = 1024
_BLOCK = _ROWS * 128


def _dropout_body(idx_ref, x_ref, oi_ref, o_ref):
    oi_ref[...] = idx_ref[...]

    base = pl.program_id(0) * _BLOCK
    idx = (
        base
        + 128 * lax.broadcasted_iota(jnp.int32, (_ROWS, 128), 0)
        + lax.broadcasted_iota(jnp.int32, (_ROWS, 128), 1)
    )
    ks = (jnp.uint32(_KS0), jnp.uint32(_KS1), jnp.uint32(_KS2))
    x0 = jnp.full((_ROWS, 128), _KS0, jnp.uint32)
    x1 = idx.astype(jnp.uint32) + ks[1]
    for i in range(5):
        for r in _ROTS[i % 2]:
            x0 = x0 + x1
            x1 = (x1 << jnp.uint32(r)) | (x1 >> jnp.uint32(32 - r))
            x1 = x1 ^ x0
        x0 = x0 + ks[(i + 1) % 3]
        x1 = x1 + ks[(i + 2) % 3] + jnp.uint32(i + 1)
    keep = (x0 ^ x1) >= jnp.uint32(0x80000000)
    x = x_ref[...].reshape(_ROWS, 128)
    out = jnp.where(keep, x * jnp.float32(2.0), jnp.float32(0.0))
    o_ref[...] = out.reshape(_BLOCK)


def kernel(x_indices, x_values):
    n = x_values.shape[0]
    xi_flat = x_indices.reshape(-1)
    oi, out = pl.pallas_call(
        _dropout_body,
        grid=(pl.cdiv(n, _BLOCK),),
        in_specs=[
            pl.BlockSpec((2 * _BLOCK,), lambda i: (i,)),
            pl.BlockSpec((_BLOCK,), lambda i: (i,)),
        ],
        out_specs=[
            pl.BlockSpec((2 * _BLOCK,), lambda i: (i,)),
            pl.BlockSpec((_BLOCK,), lambda i: (i,)),
        ],
        out_shape=[
            jax.ShapeDtypeStruct(xi_flat.shape, xi_flat.dtype),
            jax.ShapeDtypeStruct((n,), jnp.float32),
        ],
    )(xi_flat, x_values)
    return (oi.reshape(x_indices.shape), out)


# speedup vs baseline: 20.5463x; 20.5463x over previous
import jax
import jax.numpy as jnp
from jax import lax
from jax.experimental import pallas as pl
from jax.experimental.pallas import tpu as pltpu

_KS0 = 0
_KS1 = 12345
_KS2 = _KS0 ^ _KS1 ^ 0x1BD11BDA
_ROTS = ((13, 15, 26, 6), (17, 29, 16, 24))

_ROWS = 1024
_BLOCK = _ROWS * 128


def _dropout_body(idx_ref, x_ref, oi_ref, o_ref):
    oi_ref[...] = idx_ref[...]

    base = pl.program_id(0) * _BLOCK
    idx = (
        base
        + 128 * lax.broadcasted_iota(jnp.int32, (_ROWS, 128), 0)
        + lax.broadcasted_iota(jnp.int32, (_ROWS, 128), 1)
    )
    ks = (jnp.uint32(_KS0), jnp.uint32(_KS1), jnp.uint32(_KS2))
    x0 = jnp.full((_ROWS, 128), _KS0, jnp.uint32)
    x1 = idx.astype(jnp.uint32) + ks[1]
    for i in range(5):
        for r in _ROTS[i % 2]:
            x0 = x0 + x1
            x1 = (x1 << jnp.uint32(r)) | (x1 >> jnp.uint32(32 - r))
            x1 = x1 ^ x0
        x0 = x0 + ks[(i + 1) % 3]
        x1 = x1 + ks[(i + 2) % 3] + jnp.uint32(i + 1)
    keep = (x0 ^ x1) >= jnp.uint32(0x80000000)
    x = x_ref[...].reshape(_ROWS, 128)
    out = jnp.where(keep, x * jnp.float32(2.0), jnp.float32(0.0))
    o_ref[...] = out.reshape(_BLOCK)


def kernel(x_indices, x_values):
    n = x_values.shape[0]
    oi, out = pl.pallas_call(
        _dropout_body,
        grid=(pl.cdiv(n, _BLOCK),),
        in_specs=[
            pl.BlockSpec((2, _BLOCK), lambda i: (0, i)),
            pl.BlockSpec((_BLOCK,), lambda i: (i,)),
        ],
        out_specs=[
            pl.BlockSpec((2, _BLOCK), lambda i: (0, i)),
            pl.BlockSpec((_BLOCK,), lambda i: (i,)),
        ],
        out_shape=[
            jax.ShapeDtypeStruct(x_indices.shape, x_indices.dtype),
            jax.ShapeDtypeStruct((n,), jnp.float32),
        ],
    )(x_indices, x_values)
    return (oi, out)
